# baseline (speedup 1.0000x reference)
import jax
import jax.numpy as jnp
from jax.experimental import pallas as pl
from jax.experimental.pallas import tpu as pltpu

_EMB = 2048
_NE = 64
_TOK = 16384
_COEFF = 0.001
_BT = 1024
_HALF = _TOK // 2


def _router_kernel(xa_ref, xb_ref, w_ref, b_ref, probs_ref, zpart_ref):
    def head(x_ref):
        logits = jax.lax.dot_general(
            x_ref[...].astype(jnp.bfloat16), w_ref[...],
            dimension_numbers=(((1,), (1,)), ((), ())),
            preferred_element_type=jnp.float32,
        ) + b_ref[...]
        m = jnp.max(logits, axis=-1, keepdims=True)
        e = jnp.exp(logits - m)
        s = jnp.sum(e, axis=-1, keepdims=True)
        return e / s, jnp.sum(logits * logits)

    pa, za = head(xa_ref)
    pb, zb = head(xb_ref)
    probs_ref[0] = pa
    probs_ref[1] = pb
    zpart_ref[...] = (za + zb).reshape(1, 1, 1)


def kernel(x, W, b):
    nblk = _HALF // _BT
    probs2, zpart = pl.pallas_call(
        _router_kernel,
        grid=(nblk,),
        in_specs=[
            pl.BlockSpec((_BT, _EMB), lambda i: (i, 0)),
            pl.BlockSpec((_BT, _EMB), lambda i: (i + nblk, 0)),
            pl.BlockSpec((_NE, _EMB), lambda i: (0, 0)),
            pl.BlockSpec((1, _NE), lambda i: (0, 0)),
        ],
        out_specs=[
            pl.BlockSpec((2, _BT, _NE), lambda i: (0, i, 0)),
            pl.BlockSpec((1, 1, 1), lambda i: (i, 0, 0)),
        ],
        out_shape=[
            jax.ShapeDtypeStruct((2, _HALF, _NE), jnp.float32),
            jax.ShapeDtypeStruct((nblk, 1, 1), jnp.float32),
        ],
        compiler_params=pltpu.CompilerParams(
            dimension_semantics=("arbitrary",),
        ),
    )(x, x, W.astype(jnp.bfloat16), b.reshape(1, _NE))
    z_loss = jnp.sum(zpart) * (_COEFF / (_TOK * _NE))
    return (probs2.reshape(_TOK, _NE), z_loss)


# speedup vs baseline: 1.0328x; 1.0328x over previous
import jax
import jax.numpy as jnp
from jax.experimental import pallas as pl
from jax.experimental.pallas import tpu as pltpu

_EMB = 2048
_NE = 64
_TOK = 16384
_COEFF = 0.001
_BT = 512
_K = 4


def _copy_in(x_hbm, xbuf, sems, blk, slot):
    pltpu.make_async_copy(
        x_hbm.at[pl.ds(blk * _BT, _BT), :],
        xbuf.at[slot],
        sems.at[slot],
    ).start()


def _router_kernel(x_hbm, w_ref, b_ref, probs_ref, zpart_ref, xbuf, sems):
    i = pl.program_id(0)
    nblk = _TOK // _BT

    @pl.when(i == 0)
    def _prologue():
        for s in range(_K):
            _copy_in(x_hbm, xbuf, sems, s, s)

    slot = jax.lax.rem(i, _K)
    pltpu.make_async_copy(
        x_hbm.at[pl.ds(i * _BT, _BT), :],
        xbuf.at[slot],
        sems.at[slot],
    ).wait()

    logits = jax.lax.dot_general(
        xbuf[slot], w_ref[...],
        dimension_numbers=(((1,), (1,)), ((), ())),
        preferred_element_type=jnp.float32,
    ) + b_ref[...]
    m = jnp.max(logits, axis=-1, keepdims=True)
    e = jnp.exp(logits - m)
    s = jnp.sum(e, axis=-1, keepdims=True)
    probs_ref[...] = e / s
    zpart_ref[...] = jnp.sum(logits * logits).reshape(1, 1, 1)

    nxt = i + _K

    @pl.when(nxt < nblk)
    def _refill():
        _copy_in(x_hbm, xbuf, sems, nxt, slot)


def kernel(x, W, b):
    nblk = _TOK // _BT
    probs, zpart = pl.pallas_call(
        _router_kernel,
        grid=(nblk,),
        in_specs=[
            pl.BlockSpec(memory_space=pltpu.MemorySpace.HBM),
            pl.BlockSpec((_NE, _EMB), lambda i: (0, 0)),
            pl.BlockSpec((1, _NE), lambda i: (0, 0)),
        ],
        out_specs=[
            pl.BlockSpec((_BT, _NE), lambda i: (i, 0)),
            pl.BlockSpec((1, 1, 1), lambda i: (i, 0, 0)),
        ],
        out_shape=[
            jax.ShapeDtypeStruct((_TOK, _NE), jnp.float32),
            jax.ShapeDtypeStruct((nblk, 1, 1), jnp.float32),
        ],
        scratch_shapes=[
            pltpu.VMEM((_K, _BT, _EMB), jnp.float32),
            pltpu.SemaphoreType.DMA((_K,)),
        ],
        compiler_params=pltpu.CompilerParams(
            dimension_semantics=("arbitrary",),
        ),
    )(x, W, b.reshape(1, _NE))
    z_loss = jnp.sum(zpart) * (_COEFF / (_TOK * _NE))
    return (probs, z_loss)


# speedup vs baseline: 1.0374x; 1.0044x over previous
import jax
import jax.numpy as jnp
from jax.experimental import pallas as pl
from jax.experimental.pallas import tpu as pltpu

_EMB = 2048
_NE = 64
_TOK = 16384
_COEFF = 0.001
_BT = 512
_K = 4


def _copy_in(x_hbm, xbuf, sems, blk, slot):
    pltpu.make_async_copy(
        x_hbm.at[pl.ds(blk * _BT, _BT), :],
        xbuf.at[slot],
        sems.at[slot],
    ).start()


def _router_kernel(x_hbm, w_ref, b_ref, probs_ref, zpart_ref, xbuf, sems):
    i = pl.program_id(0)
    nblk = _TOK // _BT

    @pl.when(i == 0)
    def _prologue():
        for s in range(_K):
            _copy_in(x_hbm, xbuf, sems, s, s)

    slot = jax.lax.rem(i, _K)
    pltpu.make_async_copy(
        x_hbm.at[pl.ds(i * _BT, _BT), :],
        xbuf.at[slot],
        sems.at[slot],
    ).wait()

    logits = jax.lax.dot_general(
        xbuf[slot], w_ref[...],
        dimension_numbers=(((1,), (1,)), ((), ())),
        preferred_element_type=jnp.float32,
    ) + b_ref[...]
    m = jnp.max(logits, axis=-1, keepdims=True)
    e = jnp.exp(logits - m)
    s = jnp.sum(e, axis=-1, keepdims=True)
    probs_ref[...] = e / s
    zpart_ref[...] = jnp.sum(logits * logits).reshape(1, 1, 1)

    nxt = i + _K
    for sl in range(_K):
        @pl.when(jnp.logical_and(nxt < nblk, slot == sl))
        def _refill(sl=sl):
            _copy_in(x_hbm, xbuf, sems, nxt, sl)


def kernel(x, W, b):
    nblk = _TOK // _BT
    probs, zpart = pl.pallas_call(
        _router_kernel,
        grid=(nblk,),
        in_specs=[
            pl.BlockSpec(memory_space=pltpu.MemorySpace.HBM),
            pl.BlockSpec((_NE, _EMB), lambda i: (0, 0)),
            pl.BlockSpec((1, _NE), lambda i: (0, 0)),
        ],
        out_specs=[
            pl.BlockSpec((_BT, _NE), lambda i: (i, 0)),
            pl.BlockSpec((1, 1, 1), lambda i: (i, 0, 0)),
        ],
        out_shape=[
            jax.ShapeDtypeStruct((_TOK, _NE), jnp.float32),
            jax.ShapeDtypeStruct((nblk, 1, 1), jnp.float32),
        ],
        scratch_shapes=[
            pltpu.VMEM((_K, _BT, _EMB), jnp.float32),
            pltpu.SemaphoreType.DMA((_K,)),
        ],
        compiler_params=pltpu.CompilerParams(
            dimension_semantics=("arbitrary",),
        ),
    )(x, W, b.reshape(1, _NE))
    z_loss = jnp.sum(zpart) * (_COEFF / (_TOK * _NE))
    return (probs, z_loss)
